# baseline (speedup 1.0000x reference)
import functools

import jax
import jax.numpy as jnp
from jax import lax
from jax.experimental import pallas as pl
from jax.experimental.pallas import tpu as pltpu
from jax.experimental.pallas import tpu_sc as plsc

H = 64
NB = 17
B = 16384
V = 2 ** NB

_info = plsc.get_sparse_core_info()
_NC, _NS, _L = _info.num_cores, _info.num_subcores, _info.num_lanes
_NW = _NC * _NS
_BPW = B // _NW
_CHUNK = 128
_NCHUNK = _BPW // _CHUNK
_RPW = V // _NW
_BLK = 128
_NBLK = _RPW // _BLK
_NBUF = 2


def _transpose_body(embt_hbm, tsp_hbm, in_v, out_v, sems):
    wid = lax.axis_index("s") * _NC + lax.axis_index("c")
    r_base = wid * _RPW

    lanes = lax.iota(jnp.int32, _L)

    def stage_in(b, kk):
        return pltpu.async_copy(
            embt_hbm.at[:, pl.ds(pl.multiple_of(r_base + kk * _BLK, _BLK),
                                 _BLK)],
            in_v.at[b],
            sems.at[b],
        )

    for b in range(_NBUF):
        stage_in(b, b)

    def block(kk, carry):
        for b in range(_NBUF):
            k = kk * _NBUF + b
            pltpu.make_async_copy(
                embt_hbm.at[:, pl.ds(0, _BLK)], in_v.at[b], sems.at[b]
            ).wait()
            for q in range(_BLK // 2):
                for g in range(_BLK // _L):
                    src_row = lanes + (g % 4) * _L
                    src_col = jnp.full((_L,), 2 * q + (g // 4), jnp.int32)
                    v = plsc.load_gather(in_v.at[b], [src_row, src_col])
                    out_v[b, q, pl.ds(g * _L, _L)] = v

            @pl.when(k + _NBUF < _NBLK)
            def _():
                stage_in(b, k + _NBUF)

            pltpu.sync_copy(
                out_v.at[b],
                tsp_hbm.at[pl.ds(
                    pl.multiple_of((r_base + k * _BLK) // 2, _BLK // 2),
                    _BLK // 2)],
            )
        return carry

    lax.fori_loop(0, _NBLK // _NBUF, block, 0)


def _lookup_body(statest_hbm, tsp_hbm, outt_hbm, st_v, idxhi_v, half_v,
                 rows_v, selt_v, sem, gsem):
    wid = lax.axis_index("s") * _NC + lax.axis_index("c")
    base = wid * _BPW

    pltpu.sync_copy(
        statest_hbm.at[:, pl.ds(pl.multiple_of(base, _BPW), _BPW)], st_v)

    lanes = lax.iota(jnp.int32, _L)
    gpc = _CHUNK // _L

    copies = []
    for c in range(_NCHUNK):
        for g in range(gpc):
            s0 = c * _CHUNK + g * _L
            acc = jnp.zeros((_L,), jnp.int32)
            for j in range(NB):
                acc = acc + st_v[j, pl.ds(s0, _L)] * (1 << j)
            idxhi_v[c, pl.ds(g * _L, _L)] = acc >> 1
            half_v[c, pl.ds(g * _L, _L)] = (acc & 1) * H
        copies.append(
            pltpu.async_copy(
                tsp_hbm.at[idxhi_v.at[c]],
                rows_v.at[pl.ds(c * _CHUNK, _CHUNK)],
                gsem,
            )
        )

    def select_group(g, _):
        srow = g * _L + lanes
        h = half_v[g // gpc, pl.ds((g % gpc) * _L, _L)]
        for j in range(H):
            v = plsc.load_gather(rows_v, [srow, h + j])
            selt_v[j, pl.ds(g * _L, _L)] = v
        return _

    for c in range(_NCHUNK):
        copies[c].wait()
        lax.fori_loop(c * gpc, (c + 1) * gpc, select_group, 0)

    pltpu.sync_copy(
        selt_v, outt_hbm.at[:, pl.ds(pl.multiple_of(base, _BPW), _BPW)])


@jax.jit
def kernel(states, emb):
    mesh = plsc.VectorSubcoreMesh(core_axis_name="c", subcore_axis_name="s")
    params = pltpu.CompilerParams(
        needs_layout_passes=False, use_tc_tiling_on_sc=True
    )
    transpose = functools.partial(
        pl.kernel,
        mesh=mesh,
        out_type=jax.ShapeDtypeStruct((V // 2, 2 * H), jnp.float32),
        compiler_params=params,
        scratch_types=[
            pltpu.VMEM((_NBUF, H, _BLK), jnp.float32),
            pltpu.VMEM((_NBUF, _BLK // 2, 2 * H), jnp.float32),
            pltpu.SemaphoreType.DMA((_NBUF,)),
        ],
    )(_transpose_body)
    lookup = functools.partial(
        pl.kernel,
        mesh=mesh,
        out_type=jax.ShapeDtypeStruct((H, B), jnp.float32),
        compiler_params=params,
        scratch_types=[
            pltpu.VMEM((NB, _BPW), jnp.int32),
            pltpu.VMEM((_NCHUNK, _CHUNK), jnp.int32),
            pltpu.VMEM((_NCHUNK, _CHUNK), jnp.int32),
            pltpu.VMEM((_BPW, 2 * H), jnp.float32),
            pltpu.VMEM((H, _BPW), jnp.float32),
            pltpu.SemaphoreType.DMA,
            pltpu.SemaphoreType.DMA,
        ],
    )(_lookup_body)

    tsp = transpose(emb.T)
    outt = lookup(states.T, tsp)
    return outt.T


# speedup vs baseline: 1.4728x; 1.4728x over previous
import functools

import jax
import jax.numpy as jnp
from jax import lax
from jax.experimental import pallas as pl
from jax.experimental.pallas import tpu as pltpu
from jax.experimental.pallas import tpu_sc as plsc

H = 64
NB = 17
B = 16384
V = 2 ** NB

_info = plsc.get_sparse_core_info()
_NC, _NS, _L = _info.num_cores, _info.num_subcores, _info.num_lanes
_NW = _NC * _NS
_BPW = B // _NW
_CHUNK = 128
_NCHUNK = _BPW // _CHUNK
_RPW = V // _NW
_BLK = 128
_NBLK = _RPW // _BLK
_NBUF = 2


def _transpose_body(embt_hbm, tsp_hbm, in_v, out_v, sems):
    wid = lax.axis_index("s") * _NC + lax.axis_index("c")
    r_base = wid * _RPW

    lanes = lax.iota(jnp.int32, _L)

    def stage_in(b, kk):
        return pltpu.async_copy(
            embt_hbm.at[:, pl.ds(pl.multiple_of(r_base + kk * _BLK, _BLK),
                                 _BLK)],
            in_v.at[b],
            sems.at[b],
        )

    for b in range(_NBUF):
        stage_in(b, b)

    def block(kk, carry):
        for b in range(_NBUF):
            k = kk * _NBUF + b
            pltpu.make_async_copy(
                embt_hbm.at[:, pl.ds(0, _BLK)], in_v.at[b], sems.at[b]
            ).wait()
            for q0 in range(0, _BLK // 2, 2):
                vals = []
                for q in (q0, q0 + 1):
                    for g in range(_BLK // _L):
                        src_row = lanes + (g % 4) * _L
                        src_col = jnp.full((_L,), 2 * q + (g // 4), jnp.int32)
                        vals.append(
                            plsc.load_gather(in_v.at[b], [src_row, src_col]))
                i = 0
                for q in (q0, q0 + 1):
                    for g in range(_BLK // _L):
                        out_v[b, q, pl.ds(g * _L, _L)] = vals[i]
                        i += 1

            @pl.when(k + _NBUF < _NBLK)
            def _():
                stage_in(b, k + _NBUF)

            pltpu.sync_copy(
                out_v.at[b],
                tsp_hbm.at[pl.ds(
                    pl.multiple_of((r_base + k * _BLK) // 2, _BLK // 2),
                    _BLK // 2)],
            )
        return carry

    lax.fori_loop(0, _NBLK // _NBUF, block, 0)


def _lookup_body(statest_hbm, tsp_hbm, outt_hbm, st_v, idxhi_v, half_v,
                 rows_v, selt_v, sem, gsem):
    wid = lax.axis_index("s") * _NC + lax.axis_index("c")
    base = wid * _BPW

    pltpu.sync_copy(
        statest_hbm.at[:, pl.ds(pl.multiple_of(base, _BPW), _BPW)], st_v)

    lanes = lax.iota(jnp.int32, _L)
    gpc = _CHUNK // _L

    copies = []
    for c in range(_NCHUNK):
        for g in range(gpc):
            s0 = c * _CHUNK + g * _L
            acc = jnp.zeros((_L,), jnp.int32)
            for j in range(NB):
                acc = acc + st_v[j, pl.ds(s0, _L)] * (1 << j)
            idxhi_v[c, pl.ds(g * _L, _L)] = acc >> 1
            half_v[c, pl.ds(g * _L, _L)] = (acc & 1) * H
        copies.append(
            pltpu.async_copy(
                tsp_hbm.at[idxhi_v.at[c]],
                rows_v.at[pl.ds(c * _CHUNK, _CHUNK)],
                gsem,
            )
        )

    def select_group(g, _):
        srow = g * _L + lanes
        h = half_v[g // gpc, pl.ds((g % gpc) * _L, _L)]
        for j0 in range(0, H, 16):
            vals = [plsc.load_gather(rows_v, [srow, h + (j0 + jj)])
                    for jj in range(16)]
            for jj in range(16):
                selt_v[j0 + jj, pl.ds(g * _L, _L)] = vals[jj]
        return _

    for c in range(_NCHUNK):
        copies[c].wait()
        lax.fori_loop(c * gpc, (c + 1) * gpc, select_group, 0)

    pltpu.sync_copy(
        selt_v, outt_hbm.at[:, pl.ds(pl.multiple_of(base, _BPW), _BPW)])


@jax.jit
def kernel(states, emb):
    mesh = plsc.VectorSubcoreMesh(core_axis_name="c", subcore_axis_name="s")
    params = pltpu.CompilerParams(
        needs_layout_passes=False, use_tc_tiling_on_sc=True
    )
    transpose = functools.partial(
        pl.kernel,
        mesh=mesh,
        out_type=jax.ShapeDtypeStruct((V // 2, 2 * H), jnp.float32),
        compiler_params=params,
        scratch_types=[
            pltpu.VMEM((_NBUF, H, _BLK), jnp.float32),
            pltpu.VMEM((_NBUF, _BLK // 2, 2 * H), jnp.float32),
            pltpu.SemaphoreType.DMA((_NBUF,)),
        ],
    )(_transpose_body)
    lookup = functools.partial(
        pl.kernel,
        mesh=mesh,
        out_type=jax.ShapeDtypeStruct((H, B), jnp.float32),
        compiler_params=params,
        scratch_types=[
            pltpu.VMEM((NB, _BPW), jnp.int32),
            pltpu.VMEM((_NCHUNK, _CHUNK), jnp.int32),
            pltpu.VMEM((_NCHUNK, _CHUNK), jnp.int32),
            pltpu.VMEM((_BPW, 2 * H), jnp.float32),
            pltpu.VMEM((H, _BPW), jnp.float32),
            pltpu.SemaphoreType.DMA,
            pltpu.SemaphoreType.DMA,
        ],
    )(_lookup_body)

    tsp = transpose(emb.T)
    outt = lookup(states.T, tsp)
    return outt.T


# speedup vs baseline: 1.5674x; 1.0642x over previous
import functools

import jax
import jax.numpy as jnp
from jax import lax
from jax.experimental import pallas as pl
from jax.experimental.pallas import tpu as pltpu
from jax.experimental.pallas import tpu_sc as plsc

H = 64
NB = 17
B = 16384
V = 2 ** NB

_info = plsc.get_sparse_core_info()
_NC, _NS, _L = _info.num_cores, _info.num_subcores, _info.num_lanes
_NW = _NC * _NS
_BPW = B // _NW
_CHUNK = 128
_NCHUNK = _BPW // _CHUNK
_RPW = V // _NW
_SBLK = 256
_NSBLK = _RPW // _SBLK
_NBUF = 2


def _transpose_body(embt_hbm, tsp_hbm, in_v, out_v, insems, outsems):
    wid = lax.axis_index("s") * _NC + lax.axis_index("c")
    r_base = wid * _RPW

    lanes = lax.iota(jnp.int32, _L)

    def stage_in(b, kk):
        return pltpu.async_copy(
            embt_hbm.at[:, pl.ds(pl.multiple_of(r_base + kk * _SBLK, _SBLK),
                                 _SBLK)],
            in_v.at[b],
            insems.at[b],
        )

    for b in range(_NBUF):
        stage_in(b, b)

    def block(kk, carry):
        for b in range(_NBUF):
            k = kk * _NBUF + b

            @pl.when(k >= _NBUF)
            def _():
                pltpu.make_async_copy(
                    out_v.at[b], tsp_hbm.at[pl.ds(0, _SBLK // 2)],
                    outsems.at[b],
                ).wait()

            pltpu.make_async_copy(
                embt_hbm.at[:, pl.ds(0, _SBLK)], in_v.at[b], insems.at[b]
            ).wait()
            def pquad(pq, carry2):
                vals = []
                for dp in range(4):
                    p = pq * 4 + dp
                    for g in range(2 * H // _L):
                        src_row = lanes + (g % 4) * _L
                        src_col = jnp.full((_L,), 2 * p + (g // 4), jnp.int32)
                        vals.append(
                            plsc.load_gather(in_v.at[b], [src_row, src_col]))
                i = 0
                for dp in range(4):
                    p = pq * 4 + dp
                    for g in range(2 * H // _L):
                        out_v[b, p, pl.ds(g * _L, _L)] = vals[i]
                        i += 1
                return carry2

            lax.fori_loop(0, _SBLK // 8, pquad, 0)

            pltpu.async_copy(
                out_v.at[b],
                tsp_hbm.at[pl.ds(
                    pl.multiple_of((r_base + k * _SBLK) // 2, _SBLK // 2),
                    _SBLK // 2)],
                outsems.at[b],
            )

            @pl.when(k + _NBUF < _NSBLK)
            def _():
                stage_in(b, k + _NBUF)

        return carry

    lax.fori_loop(0, _NSBLK // _NBUF, block, 0)

    for b in range(_NBUF):
        pltpu.make_async_copy(
            out_v.at[b], tsp_hbm.at[pl.ds(0, _SBLK // 2)], outsems.at[b]
        ).wait()


def _lookup_body(statest_hbm, tsp_hbm, outt_hbm, st_v, idxhi_v, half_v,
                 rows_v, selt_v, sem, gsem):
    wid = lax.axis_index("s") * _NC + lax.axis_index("c")
    base = wid * _BPW

    pltpu.sync_copy(
        statest_hbm.at[:, pl.ds(pl.multiple_of(base, _BPW), _BPW)], st_v)

    lanes = lax.iota(jnp.int32, _L)
    gpc = _CHUNK // _L

    copies = []
    for c in range(_NCHUNK):
        for g in range(gpc):
            s0 = c * _CHUNK + g * _L
            acc = jnp.zeros((_L,), jnp.int32)
            for j in range(NB):
                acc = acc + st_v[j, pl.ds(s0, _L)] * (1 << j)
            idxhi_v[c, pl.ds(g * _L, _L)] = acc >> 1
            half_v[c, pl.ds(g * _L, _L)] = (acc & 1) * H
        copies.append(
            pltpu.async_copy(
                tsp_hbm.at[idxhi_v.at[c]],
                rows_v.at[pl.ds(c * _CHUNK, _CHUNK)],
                gsem,
            )
        )

    def select_group(g, _):
        srow = g * _L + lanes
        h = half_v[g // gpc, pl.ds((g % gpc) * _L, _L)]
        for j0 in range(0, H, 16):
            vals = [plsc.load_gather(rows_v, [srow, h + (j0 + jj)])
                    for jj in range(16)]
            for jj in range(16):
                selt_v[j0 + jj, pl.ds(g * _L, _L)] = vals[jj]
        return _

    for c in range(_NCHUNK):
        copies[c].wait()
        lax.fori_loop(c * gpc, (c + 1) * gpc, select_group, 0)

    pltpu.sync_copy(
        selt_v, outt_hbm.at[:, pl.ds(pl.multiple_of(base, _BPW), _BPW)])


@jax.jit
def kernel(states, emb):
    mesh = plsc.VectorSubcoreMesh(core_axis_name="c", subcore_axis_name="s")
    params = pltpu.CompilerParams(
        needs_layout_passes=False, use_tc_tiling_on_sc=True
    )
    transpose = functools.partial(
        pl.kernel,
        mesh=mesh,
        out_type=jax.ShapeDtypeStruct((V // 2, 2 * H), jnp.float32),
        compiler_params=params,
        scratch_types=[
            pltpu.VMEM((_NBUF, H, _SBLK), jnp.float32),
            pltpu.VMEM((_NBUF, _SBLK // 2, 2 * H), jnp.float32),
            pltpu.SemaphoreType.DMA((_NBUF,)),
            pltpu.SemaphoreType.DMA((_NBUF,)),
        ],
    )(_transpose_body)
    lookup = functools.partial(
        pl.kernel,
        mesh=mesh,
        out_type=jax.ShapeDtypeStruct((H, B), jnp.float32),
        compiler_params=params,
        scratch_types=[
            pltpu.VMEM((NB, _BPW), jnp.int32),
            pltpu.VMEM((_NCHUNK, _CHUNK), jnp.int32),
            pltpu.VMEM((_NCHUNK, _CHUNK), jnp.int32),
            pltpu.VMEM((_BPW, 2 * H), jnp.float32),
            pltpu.VMEM((H, _BPW), jnp.float32),
            pltpu.SemaphoreType.DMA,
            pltpu.SemaphoreType.DMA,
        ],
    )(_lookup_body)

    tsp = transpose(emb.T)
    outt = lookup(states.T, tsp)
    return outt.T


# speedup vs baseline: 4.6277x; 2.9524x over previous
import functools

import jax
import jax.numpy as jnp
import numpy as np
from jax import lax
from jax.experimental import pallas as pl
from jax.experimental.pallas import tpu as pltpu
from jax.experimental.pallas import tpu_sc as plsc

H = 64
NB = 17
B = 16384
V = 2 ** NB

_info = plsc.get_sparse_core_info()
_NC, _NS, _L = _info.num_cores, _info.num_subcores, _info.num_lanes
_NW = _NC * _NS
_BPW = B // _NW
_CHUNK = 128
_NCHUNK = _BPW // _CHUNK
_RPW = V // _NW
_SBLK = 256
_NSBLK = _RPW // _SBLK
_NBUF = 2


def _transpose_body(embt_hbm, tsp_hbm, in_v, out_v, insems, outsems):
    wid = lax.axis_index("s") * _NC + lax.axis_index("c")
    r_base = wid * _RPW

    lanes = lax.iota(jnp.int32, _L)

    def stage_in(b, kk):
        return pltpu.async_copy(
            embt_hbm.at[:, pl.ds(pl.multiple_of(r_base + kk * _SBLK, _SBLK),
                                 _SBLK)],
            in_v.at[b],
            insems.at[b],
        )

    for b in range(_NBUF):
        stage_in(b, b)

    def block(kk, carry):
        for b in range(_NBUF):
            k = kk * _NBUF + b

            @pl.when(k >= _NBUF)
            def _():
                pltpu.make_async_copy(
                    out_v.at[b], tsp_hbm.at[pl.ds(0, _SBLK // 2)],
                    outsems.at[b],
                ).wait()

            pltpu.make_async_copy(
                embt_hbm.at[:, pl.ds(0, _SBLK)], in_v.at[b], insems.at[b]
            ).wait()
            def iblock(ib, carry2):
                i0 = ib * _L
                p0 = ib * (_L // 2)
                for x0 in range(0, H, _L):
                    jr = lanes
                    for k0 in range(0, _L, 8):
                        dst_rows = []
                        dst_cols = []
                        vals = []
                        for k in range(k0, k0 + 8):
                            src_row = lanes + x0
                            src_col = jr + i0
                            dst_row = (jr >> 1) + p0
                            dst_col = (jr & 1) * H + x0 + lanes
                            vals.append(plsc.load_gather(
                                in_v.at[b], [src_row, src_col]))
                            dst_rows.append(dst_row)
                            dst_cols.append(dst_col)
                            jr = (jr + 1) & (_L - 1)
                        for kk in range(8):
                            plsc.store_scatter(
                                out_v.at[b],
                                [dst_rows[kk], dst_cols[kk]],
                                vals[kk])
                return carry2

            lax.fori_loop(0, _SBLK // _L, iblock, 0)

            pltpu.async_copy(
                out_v.at[b],
                tsp_hbm.at[pl.ds(
                    pl.multiple_of((r_base + k * _SBLK) // 2, _SBLK // 2),
                    _SBLK // 2)],
                outsems.at[b],
            )

            @pl.when(k + _NBUF < _NSBLK)
            def _():
                stage_in(b, k + _NBUF)

        return carry

    lax.fori_loop(0, _NSBLK // _NBUF, block, 0)

    for b in range(_NBUF):
        pltpu.make_async_copy(
            out_v.at[b], tsp_hbm.at[pl.ds(0, _SBLK // 2)], outsems.at[b]
        ).wait()


def _lookup_body(statest_hbm, tsp_hbm, outt_hbm, st_v, idxhi_v, half_v,
                 rows_v, selt_v, sem, gsem):
    wid = lax.axis_index("s") * _NC + lax.axis_index("c")
    base = wid * _BPW

    pltpu.sync_copy(
        statest_hbm.at[:, pl.ds(pl.multiple_of(base, _BPW), _BPW)], st_v)

    lanes = lax.iota(jnp.int32, _L)
    gpc = _CHUNK // _L

    copies = []
    for c in range(_NCHUNK):
        for g in range(gpc):
            s0 = c * _CHUNK + g * _L
            acc = jnp.zeros((_L,), jnp.int32)
            for j in range(NB):
                acc = acc + st_v[j, pl.ds(s0, _L)] * (1 << j)
            idxhi_v[c, pl.ds(g * _L, _L)] = acc >> 1
            half_v[c, pl.ds(g * _L, _L)] = (acc & 1) * H
        copies.append(
            pltpu.async_copy(
                tsp_hbm.at[idxhi_v.at[c]],
                rows_v.at[pl.ds(c * _CHUNK, _CHUNK)],
                gsem,
            )
        )

    rzero = st_v[0, pl.ds(0, _L)] >> 1
    rlanes = lanes + rzero

    def select_group(g, _):
        srow = g * _L + rlanes
        h = half_v[g // gpc, pl.ds((g % gpc) * _L, _L)]
        jr = rlanes
        for k0 in range(0, H, 8):
            vals = []
            jrots = []
            for k in range(k0, k0 + 8):
                jrots.append(jr)
                vals.append(plsc.load_gather(rows_v, [srow, h + jr]))
                jr = (jr + 1) & (H - 1)
            for kk in range(8):
                plsc.store_scatter(selt_v, [jrots[kk], srow], vals[kk])
        return _

    for c in range(_NCHUNK):
        copies[c].wait()
        lax.fori_loop(c * gpc, (c + 1) * gpc, select_group, 0)

    pltpu.sync_copy(
        selt_v, outt_hbm.at[:, pl.ds(pl.multiple_of(base, _BPW), _BPW)])


@jax.jit
def kernel(states, emb):
    mesh = plsc.VectorSubcoreMesh(core_axis_name="c", subcore_axis_name="s")
    params = pltpu.CompilerParams(
        needs_layout_passes=False, use_tc_tiling_on_sc=True
    )
    transpose = functools.partial(
        pl.kernel,
        mesh=mesh,
        out_type=jax.ShapeDtypeStruct((V // 2, 2 * H), jnp.float32),
        compiler_params=params,
        scratch_types=[
            pltpu.VMEM((_NBUF, H, _SBLK), jnp.float32),
            pltpu.VMEM((_NBUF, _SBLK // 2, 2 * H), jnp.float32),
            pltpu.SemaphoreType.DMA((_NBUF,)),
            pltpu.SemaphoreType.DMA((_NBUF,)),
        ],
    )(_transpose_body)
    lookup = functools.partial(
        pl.kernel,
        mesh=mesh,
        out_type=jax.ShapeDtypeStruct((H, B), jnp.float32),
        compiler_params=params,
        scratch_types=[
            pltpu.VMEM((NB, _BPW), jnp.int32),
            pltpu.VMEM((_NCHUNK, _CHUNK), jnp.int32),
            pltpu.VMEM((_NCHUNK, _CHUNK), jnp.int32),
            pltpu.VMEM((_BPW, 2 * H), jnp.float32),
            pltpu.VMEM((H, _BPW), jnp.float32),
            pltpu.SemaphoreType.DMA,
            pltpu.SemaphoreType.DMA,
        ],
    )(_lookup_body)

    tsp = transpose(emb.T)
    outt = lookup(states.T, tsp)
    return outt.T
